# baseline (speedup 1.0000x reference)
import functools

import jax
import jax.numpy as jnp
from jax import lax
from jax.experimental import pallas as pl
from jax.experimental.pallas import tpu as pltpu
from jax.experimental.pallas import tpu_sc as plsc

M = 100000
B = 16384
D = 256
L = 16
NC = 2
NS = 16
NW = NC * NS
TS = 3136
NV_FULL = TS // L
TAIL = M - (NW - 1) * TS
NSCAN = B // L
WC = 64
POS_BITS = 14
POS_MASK = (1 << POS_BITS) - 1
CBLK = 10000


def _cp_body(x_ref, o_ref):
  o_ref[...] = x_ref[...]


_tc_copy = pl.pallas_call(
    _cp_body,
    grid=(M // CBLK,),
    in_specs=[pl.BlockSpec((CBLK, D), lambda i: (i, 0))],
    out_specs=pl.BlockSpec((CBLK, D), lambda i: (i, 0)),
    out_shape=jax.ShapeDtypeStruct((M, D), jnp.float32),
)


def _prep_body(ids_hbm, ts_hbm,
               upd_hbm, aggts_hbm, winp_hbm, wins_hbm, cnt_hbm,
               ids_v, ts_v, table_v, upd_loc, tsl_loc, winp_v, wins_v,
               key16, sem_ids, sem_ts):
  wid = lax.axis_index("c") * NS + lax.axis_index("s")
  lo = wid * TS
  hi = lo + TS
  size = jnp.minimum(TS, M - lo)
  nv = size // L
  iota = lax.iota(jnp.int32, L)

  pltpu.async_copy(ids_hbm, ids_v, sem_ids)
  pltpu.async_copy(ts_hbm, ts_v, sem_ts)

  def init(i, _):
    table_v[pl.ds(i * L, L)] = jnp.full((L,), -1, jnp.int32)
    return 0
  lax.fori_loop(0, NV_FULL, init, 0)

  pltpu.make_async_copy(ids_hbm, ids_v, sem_ids).wait()

  def scan(i, _):
    ids = ids_v[pl.ds(i * L, L)]
    inr = (ids >= lo) & (ids < hi)

    @pl.when(jnp.any(inr))
    def _():
      poss = i * L + iota
      key = (ids << POS_BITS) + poss
      skey = jnp.sort(key)
      sid = skey >> POS_BITS
      spos = skey & POS_MASK
      key16[...] = sid
      nxt = plsc.load_gather(key16, [jnp.minimum(iota + 1, L - 1)])
      is_last = (sid != nxt) | (iota == L - 1)
      msk = is_last & (sid >= lo) & (sid < hi)
      idx = jnp.clip(sid - lo, 0, TS - 1)
      plsc.store_scatter(table_v, [idx], spos, mask=msk)
    return 0
  lax.fori_loop(0, NSCAN, scan, 0)

  pltpu.make_async_copy(ts_hbm, ts_v, sem_ts).wait()
  key16[...] = jnp.zeros((L,), jnp.int32)

  def emit(j, _):
    kvec = key16[...]
    lp = table_v[pl.ds(j * L, L)]
    upd = lp >= 0
    updi = jnp.where(upd, 1, 0)
    safe = jnp.maximum(lp, 0)
    tsg = plsc.load_gather(ts_v, [safe]) * updi.astype(jnp.float32)
    upd_loc[pl.ds(j * L, L)] = updi
    tsl_loc[pl.ds(j * L, L)] = tsg
    csum = plsc.cumsum(updi)
    offs = jnp.clip(kvec + csum - 1, 0, TS - 1)
    plsc.store_scatter(winp_v, [offs], safe, mask=upd)
    slot = lo + j * L + iota
    plsc.store_scatter(wins_v, [offs], slot, mask=upd)
    key16[...] = kvec + plsc.all_reduce_population_count(upd)
    return 0
  lax.fori_loop(0, nv, emit, 0)
  kvec = key16[...]
  k_s = kvec[0]

  @pl.when(k_s > 0)
  def _():
    kpad = ((k_s + WC - 1) // WC) * WC
    klast = jnp.full((L,), 0, jnp.int32) + (k_s - 1)
    lastw = plsc.load_gather(winp_v, [klast])
    lasts = plsc.load_gather(wins_v, [klast])

    def pad(t, _):
      idx = k_s + t * L + iota
      m = idx < kpad
      ii = jnp.clip(idx, 0, TS - 1)
      plsc.store_scatter(winp_v, [ii], lastw, mask=m)
      plsc.store_scatter(wins_v, [ii], lasts, mask=m)
      return 0
    lax.fori_loop(0, WC // L, pad, 0)

  pltpu.async_copy(winp_v, winp_hbm.at[wid], sem_ids)
  pltpu.async_copy(wins_v, wins_hbm.at[wid], sem_ts)

  def out_full():
    pltpu.sync_copy(upd_loc, upd_hbm.at[pl.ds(lo, TS)])
    pltpu.sync_copy(tsl_loc, aggts_hbm.at[pl.ds(lo, TS)])
  def out_tail():
    pltpu.sync_copy(upd_loc.at[pl.ds(0, TAIL)], upd_hbm.at[pl.ds(lo, TAIL)])
    pltpu.sync_copy(tsl_loc.at[pl.ds(0, TAIL)], aggts_hbm.at[pl.ds(lo, TAIL)])
  lax.cond(nv == NV_FULL, out_full, out_tail)

  key16[...] = kvec
  pltpu.sync_copy(key16, cnt_hbm.at[wid])
  pltpu.make_async_copy(winp_v, winp_hbm.at[wid], sem_ids).wait()
  pltpu.make_async_copy(wins_v, wins_hbm.at[wid], sem_ts).wait()


_sc_prep = pl.kernel(
    _prep_body,
    out_type=[
        jax.ShapeDtypeStruct((M,), jnp.int32),
        jax.ShapeDtypeStruct((M,), jnp.float32),
        jax.ShapeDtypeStruct((NW, TS), jnp.int32),
        jax.ShapeDtypeStruct((NW, TS), jnp.int32),
        jax.ShapeDtypeStruct((NW, L), jnp.int32),
    ],
    mesh=plsc.VectorSubcoreMesh(core_axis_name="c", subcore_axis_name="s"),
    compiler_params=pltpu.CompilerParams(needs_layout_passes=False),
    scratch_types=[
        pltpu.VMEM((B,), jnp.int32),
        pltpu.VMEM((B,), jnp.float32),
        pltpu.VMEM((TS,), jnp.int32),
        pltpu.VMEM((TS,), jnp.int32),
        pltpu.VMEM((TS,), jnp.float32),
        pltpu.VMEM((TS,), jnp.int32),
        pltpu.VMEM((TS,), jnp.int32),
        pltpu.VMEM((L,), jnp.int32),
        pltpu.SemaphoreType.DMA,
        pltpu.SemaphoreType.DMA,
    ],
)


def _scat_body(msgs_hbm, winp_hbm, wins_hbm, cnt_hbm, newmem_hbm,
               winp_v, wins_v, key16, src0_v, dst0_v, src1_v, dst1_v,
               src2_v, dst2_v, src3_v, dst3_v,
               rows0_v, rows1_v, rows2_v, rows3_v,
               g0, s0, g1, s1, g2, s2, g3, s3):
  wid = lax.axis_index("c") * NS + lax.axis_index("s")
  pltpu.async_copy(winp_hbm.at[wid], winp_v, g0)
  pltpu.async_copy(wins_hbm.at[wid], wins_v, g1)
  pltpu.sync_copy(cnt_hbm.at[wid], key16)
  k_s = key16[...][0]
  pltpu.make_async_copy(winp_hbm.at[wid], winp_v, g0).wait()
  pltpu.make_async_copy(wins_hbm.at[wid], wins_v, g1).wait()

  @pl.when(k_s > 0)
  def _():
    nch = (k_s + WC - 1) // WC

    def ldidx(c, sref, dref):
      def ld(t, _):
        gidx = c * WC + t * L + lax.iota(jnp.int32, L)
        sref[pl.ds(t * L, L)] = plsc.load_gather(winp_v, [gidx])
        dref[pl.ds(t * L, L)] = plsc.load_gather(wins_v, [gidx])
        return 0
      lax.fori_loop(0, WC // L, ld, 0)

    srefs = [src0_v, src1_v, src2_v, src3_v]
    drefs = [dst0_v, dst1_v, dst2_v, dst3_v]
    rrefs = [rows0_v, rows1_v, rows2_v, rows3_v]
    gsems = [g0, g1, g2, g3]
    ssems = [s0, s1, s2, s3]

    for i in range(4):
      @pl.when(i < nch)
      def _():
        ldidx(i, srefs[i], drefs[i])
        pltpu.async_copy(msgs_hbm.at[srefs[i]], rrefs[i], gsems[i])

    def wbody(u, _):
      cb = 4 * u
      for i in range(4):
        c = cb + i
        @pl.when(c < nch)
        def _():
          pltpu.make_async_copy(msgs_hbm.at[srefs[i]], rrefs[i],
                                gsems[i]).wait()
          pltpu.async_copy(rrefs[i], newmem_hbm.at[drefs[i]], ssems[i])
          @pl.when(c + 4 < nch)
          def _():
            pltpu.make_async_copy(rrefs[i], newmem_hbm.at[drefs[i]],
                                  ssems[i]).wait()
            ldidx(c + 4, srefs[i], drefs[i])
            pltpu.async_copy(msgs_hbm.at[srefs[i]], rrefs[i], gsems[i])
      return 0
    lax.fori_loop(0, (nch + 3) // 4, wbody, 0)
    for i in range(4):
      @pl.when(jnp.minimum(nch, 4) > i)
      def _():
        pltpu.make_async_copy(rrefs[i], newmem_hbm.at[drefs[i]],
                              ssems[i]).wait()


_sc_scatter = pl.kernel(
    _scat_body,
    out_type=[],
    mesh=plsc.VectorSubcoreMesh(core_axis_name="c", subcore_axis_name="s"),
    compiler_params=pltpu.CompilerParams(needs_layout_passes=False),
    scratch_types=[
        pltpu.VMEM((TS,), jnp.int32),
        pltpu.VMEM((TS,), jnp.int32),
        pltpu.VMEM((L,), jnp.int32),
        pltpu.VMEM((WC,), jnp.int32),
        pltpu.VMEM((WC,), jnp.int32),
        pltpu.VMEM((WC,), jnp.int32),
        pltpu.VMEM((WC,), jnp.int32),
        pltpu.VMEM((WC,), jnp.int32),
        pltpu.VMEM((WC,), jnp.int32),
        pltpu.VMEM((WC,), jnp.int32),
        pltpu.VMEM((WC,), jnp.int32),
        pltpu.VMEM((WC, D), jnp.float32),
        pltpu.VMEM((WC, D), jnp.float32),
        pltpu.VMEM((WC, D), jnp.float32),
        pltpu.VMEM((WC, D), jnp.float32),
        pltpu.SemaphoreType.DMA,
        pltpu.SemaphoreType.DMA,
        pltpu.SemaphoreType.DMA,
        pltpu.SemaphoreType.DMA,
        pltpu.SemaphoreType.DMA,
        pltpu.SemaphoreType.DMA,
        pltpu.SemaphoreType.DMA,
        pltpu.SemaphoreType.DMA,
    ],
)


def kernel(node_ids, messages, timestamps, mem):
  node_ids = node_ids.astype(jnp.int32)
  timestamps = timestamps.astype(jnp.float32)
  new_mem0 = _tc_copy(mem)
  upd, agg_ts, winp, wins, cnt = _sc_prep(node_ids, timestamps)
  r = jax.new_ref(new_mem0)
  _sc_scatter(messages, winp, wins, cnt, r)
  return r[...], upd.astype(bool), agg_ts


# speedup vs baseline: 1.0422x; 1.0422x over previous
import functools

import jax
import jax.numpy as jnp
from jax import lax
from jax.experimental import pallas as pl
from jax.experimental.pallas import tpu as pltpu
from jax.experimental.pallas import tpu_sc as plsc

M = 100000
B = 16384
D = 256
L = 16
NC = 2
NS = 16
NW = NC * NS
TS = 3136
NV_FULL = TS // L
TAIL = M - (NW - 1) * TS
NSCAN = B // L
WC = 32
POS_BITS = 14
POS_MASK = (1 << POS_BITS) - 1
CBLK = 10000


def _cp_body(x_ref, o_ref):
  o_ref[...] = x_ref[...]


_tc_copy = pl.pallas_call(
    _cp_body,
    grid=(M // CBLK,),
    in_specs=[pl.BlockSpec((CBLK, D), lambda i: (i, 0))],
    out_specs=pl.BlockSpec((CBLK, D), lambda i: (i, 0)),
    out_shape=jax.ShapeDtypeStruct((M, D), jnp.float32),
)


def _prep_body(ids_hbm, ts_hbm,
               upd_hbm, aggts_hbm, winp_hbm, wins_hbm, cnt_hbm,
               ids_v, ts_v, table_v, upd_loc, tsl_loc, winp_v, wins_v,
               key16, sem_ids, sem_ts):
  wid = lax.axis_index("c") * NS + lax.axis_index("s")
  lo = wid * TS
  hi = lo + TS
  size = jnp.minimum(TS, M - lo)
  nv = size // L
  iota = lax.iota(jnp.int32, L)

  pltpu.async_copy(ids_hbm, ids_v, sem_ids)
  pltpu.async_copy(ts_hbm, ts_v, sem_ts)

  def init(i, _):
    table_v[pl.ds(i * L, L)] = jnp.full((L,), -1, jnp.int32)
    return 0
  lax.fori_loop(0, NV_FULL, init, 0)

  pltpu.make_async_copy(ids_hbm, ids_v, sem_ids).wait()

  def scan(i, _):
    ids = ids_v[pl.ds(i * L, L)]
    inr = (ids >= lo) & (ids < hi)

    @pl.when(jnp.any(inr))
    def _():
      poss = i * L + iota
      key = (ids << POS_BITS) + poss
      skey = jnp.sort(key)
      sid = skey >> POS_BITS
      spos = skey & POS_MASK
      key16[...] = sid
      nxt = plsc.load_gather(key16, [jnp.minimum(iota + 1, L - 1)])
      is_last = (sid != nxt) | (iota == L - 1)
      msk = is_last & (sid >= lo) & (sid < hi)
      idx = jnp.clip(sid - lo, 0, TS - 1)
      plsc.store_scatter(table_v, [idx], spos, mask=msk)
    return 0
  lax.fori_loop(0, NSCAN, scan, 0)

  pltpu.make_async_copy(ts_hbm, ts_v, sem_ts).wait()
  key16[...] = jnp.zeros((L,), jnp.int32)

  def emit(j, _):
    kvec = key16[...]
    lp = table_v[pl.ds(j * L, L)]
    upd = lp >= 0
    updi = jnp.where(upd, 1, 0)
    safe = jnp.maximum(lp, 0)
    tsg = plsc.load_gather(ts_v, [safe]) * updi.astype(jnp.float32)
    upd_loc[pl.ds(j * L, L)] = updi
    tsl_loc[pl.ds(j * L, L)] = tsg
    csum = plsc.cumsum(updi)
    offs = jnp.clip(kvec + csum - 1, 0, TS - 1)
    plsc.store_scatter(winp_v, [offs], safe, mask=upd)
    slot = lo + j * L + iota
    plsc.store_scatter(wins_v, [offs], slot, mask=upd)
    key16[...] = kvec + plsc.all_reduce_population_count(upd)
    return 0
  lax.fori_loop(0, nv, emit, 0)
  kvec = key16[...]
  k_s = kvec[0]

  @pl.when(k_s > 0)
  def _():
    kpad = ((k_s + WC - 1) // WC) * WC
    klast = jnp.full((L,), 0, jnp.int32) + (k_s - 1)
    lastw = plsc.load_gather(winp_v, [klast])
    lasts = plsc.load_gather(wins_v, [klast])

    def pad(t, _):
      idx = k_s + t * L + iota
      m = idx < kpad
      ii = jnp.clip(idx, 0, TS - 1)
      plsc.store_scatter(winp_v, [ii], lastw, mask=m)
      plsc.store_scatter(wins_v, [ii], lasts, mask=m)
      return 0
    lax.fori_loop(0, WC // L, pad, 0)

  pltpu.async_copy(winp_v, winp_hbm.at[wid], sem_ids)
  pltpu.async_copy(wins_v, wins_hbm.at[wid], sem_ts)

  def out_full():
    pltpu.sync_copy(upd_loc, upd_hbm.at[pl.ds(lo, TS)])
    pltpu.sync_copy(tsl_loc, aggts_hbm.at[pl.ds(lo, TS)])
  def out_tail():
    pltpu.sync_copy(upd_loc.at[pl.ds(0, TAIL)], upd_hbm.at[pl.ds(lo, TAIL)])
    pltpu.sync_copy(tsl_loc.at[pl.ds(0, TAIL)], aggts_hbm.at[pl.ds(lo, TAIL)])
  lax.cond(nv == NV_FULL, out_full, out_tail)

  key16[...] = kvec
  pltpu.sync_copy(key16, cnt_hbm.at[wid])
  pltpu.make_async_copy(winp_v, winp_hbm.at[wid], sem_ids).wait()
  pltpu.make_async_copy(wins_v, wins_hbm.at[wid], sem_ts).wait()


_sc_prep = pl.kernel(
    _prep_body,
    out_type=[
        jax.ShapeDtypeStruct((M,), jnp.int32),
        jax.ShapeDtypeStruct((M,), jnp.float32),
        jax.ShapeDtypeStruct((NW, TS), jnp.int32),
        jax.ShapeDtypeStruct((NW, TS), jnp.int32),
        jax.ShapeDtypeStruct((NW, L), jnp.int32),
    ],
    mesh=plsc.VectorSubcoreMesh(core_axis_name="c", subcore_axis_name="s"),
    compiler_params=pltpu.CompilerParams(needs_layout_passes=False),
    scratch_types=[
        pltpu.VMEM((B,), jnp.int32),
        pltpu.VMEM((B,), jnp.float32),
        pltpu.VMEM((TS,), jnp.int32),
        pltpu.VMEM((TS,), jnp.int32),
        pltpu.VMEM((TS,), jnp.float32),
        pltpu.VMEM((TS,), jnp.int32),
        pltpu.VMEM((TS,), jnp.int32),
        pltpu.VMEM((L,), jnp.int32),
        pltpu.SemaphoreType.DMA,
        pltpu.SemaphoreType.DMA,
    ],
)


def _scat_body(msgs_hbm, winp_hbm, wins_hbm, cnt_hbm, newmem_hbm,
               winp_v, wins_v, key16, src0_v, dst0_v, src1_v, dst1_v,
               src2_v, dst2_v, src3_v, dst3_v,
               rows0_v, rows1_v, rows2_v, rows3_v,
               g0, s0, g1, s1, g2, s2, g3, s3):
  wid = lax.axis_index("c") * NS + lax.axis_index("s")
  pltpu.async_copy(winp_hbm.at[wid], winp_v, g0)
  pltpu.async_copy(wins_hbm.at[wid], wins_v, g1)
  pltpu.sync_copy(cnt_hbm.at[wid], key16)
  k_s = key16[...][0]
  pltpu.make_async_copy(winp_hbm.at[wid], winp_v, g0).wait()
  pltpu.make_async_copy(wins_hbm.at[wid], wins_v, g1).wait()

  @pl.when(k_s > 0)
  def _():
    nch = (k_s + WC - 1) // WC

    def ldidx(c, sref, dref):
      def ld(t, _):
        gidx = c * WC + t * L + lax.iota(jnp.int32, L)
        sref[pl.ds(t * L, L)] = plsc.load_gather(winp_v, [gidx])
        dref[pl.ds(t * L, L)] = plsc.load_gather(wins_v, [gidx])
        return 0
      lax.fori_loop(0, WC // L, ld, 0)

    srefs = [src0_v, src1_v, src2_v, src3_v]
    drefs = [dst0_v, dst1_v, dst2_v, dst3_v]
    rrefs = [rows0_v, rows1_v, rows2_v, rows3_v]
    gsems = [g0, g1, g2, g3]
    ssems = [s0, s1, s2, s3]

    for i in range(4):
      @pl.when(i < nch)
      def _():
        ldidx(i, srefs[i], drefs[i])
        pltpu.async_copy(msgs_hbm.at[srefs[i]], rrefs[i], gsems[i])

    def wbody(u, _):
      cb = 4 * u
      for i in range(4):
        c = cb + i
        @pl.when(c < nch)
        def _():
          pltpu.make_async_copy(msgs_hbm.at[srefs[i]], rrefs[i],
                                gsems[i]).wait()
          pltpu.async_copy(rrefs[i], newmem_hbm.at[drefs[i]], ssems[i])
          @pl.when(c + 4 < nch)
          def _():
            pltpu.make_async_copy(rrefs[i], newmem_hbm.at[drefs[i]],
                                  ssems[i]).wait()
            ldidx(c + 4, srefs[i], drefs[i])
            pltpu.async_copy(msgs_hbm.at[srefs[i]], rrefs[i], gsems[i])
      return 0
    lax.fori_loop(0, (nch + 3) // 4, wbody, 0)
    for i in range(4):
      @pl.when(jnp.minimum(nch, 4) > i)
      def _():
        pltpu.make_async_copy(rrefs[i], newmem_hbm.at[drefs[i]],
                              ssems[i]).wait()


_sc_scatter = pl.kernel(
    _scat_body,
    out_type=[],
    mesh=plsc.VectorSubcoreMesh(core_axis_name="c", subcore_axis_name="s"),
    compiler_params=pltpu.CompilerParams(needs_layout_passes=False),
    scratch_types=[
        pltpu.VMEM((TS,), jnp.int32),
        pltpu.VMEM((TS,), jnp.int32),
        pltpu.VMEM((L,), jnp.int32),
        pltpu.VMEM((WC,), jnp.int32),
        pltpu.VMEM((WC,), jnp.int32),
        pltpu.VMEM((WC,), jnp.int32),
        pltpu.VMEM((WC,), jnp.int32),
        pltpu.VMEM((WC,), jnp.int32),
        pltpu.VMEM((WC,), jnp.int32),
        pltpu.VMEM((WC,), jnp.int32),
        pltpu.VMEM((WC,), jnp.int32),
        pltpu.VMEM((WC, D), jnp.float32),
        pltpu.VMEM((WC, D), jnp.float32),
        pltpu.VMEM((WC, D), jnp.float32),
        pltpu.VMEM((WC, D), jnp.float32),
        pltpu.SemaphoreType.DMA,
        pltpu.SemaphoreType.DMA,
        pltpu.SemaphoreType.DMA,
        pltpu.SemaphoreType.DMA,
        pltpu.SemaphoreType.DMA,
        pltpu.SemaphoreType.DMA,
        pltpu.SemaphoreType.DMA,
        pltpu.SemaphoreType.DMA,
    ],
)


def kernel(node_ids, messages, timestamps, mem):
  node_ids = node_ids.astype(jnp.int32)
  timestamps = timestamps.astype(jnp.float32)
  new_mem0 = _tc_copy(mem)
  upd, agg_ts, winp, wins, cnt = _sc_prep(node_ids, timestamps)
  r = jax.new_ref(new_mem0)
  _sc_scatter(messages, winp, wins, cnt, r)
  return r[...], upd.astype(bool), agg_ts


# speedup vs baseline: 1.0445x; 1.0022x over previous
import jax
import jax.numpy as jnp
from jax import lax
from jax.experimental import pallas as pl
from jax.experimental.pallas import tpu as pltpu
from jax.experimental.pallas import tpu_sc as plsc

M = 100000
B = 16384
D = 256
L = 16
NC = 2
NS = 16
NW = NC * NS
TS = 3136
NV_FULL = TS // L
TAIL = M - (NW - 1) * TS
NSCAN = B // L
WC = 32
POS_BITS = 14
POS_MASK = (1 << POS_BITS) - 1
CBLK = 10000


def _cp_body(x_ref, o_ref):
  o_ref[...] = x_ref[...]


_tc_copy = pl.pallas_call(
    _cp_body,
    grid=(M // CBLK,),
    in_specs=[pl.BlockSpec((CBLK, D), lambda i: (i, 0))],
    out_specs=pl.BlockSpec((CBLK, D), lambda i: (i, 0)),
    out_shape=jax.ShapeDtypeStruct((M, D), jnp.float32),
)


def _prep_body(ids_hbm, ts_hbm,
               upd_hbm, aggts_hbm, winp_hbm, wins_hbm, cnt_hbm,
               ids_v, ts_v, table_v, upd_loc, tsl_loc, winp_v, wins_v,
               key16, sem_ids, sem_ts):
  wid = lax.axis_index("c") * NS + lax.axis_index("s")
  lo = wid * TS
  hi = lo + TS
  size = jnp.minimum(TS, M - lo)
  nv = size // L
  iota = lax.iota(jnp.int32, L)

  pltpu.async_copy(ids_hbm, ids_v, sem_ids)
  pltpu.async_copy(ts_hbm, ts_v, sem_ts)

  def init(i, _):
    table_v[pl.ds(i * L, L)] = jnp.full((L,), -1, jnp.int32)
    return 0
  lax.fori_loop(0, NV_FULL, init, 0)

  pltpu.make_async_copy(ids_hbm, ids_v, sem_ids).wait()

  def scan(i, _):
    ids = ids_v[pl.ds(i * L, L)]
    inr = (ids >= lo) & (ids < hi)

    @pl.when(jnp.any(inr))
    def _():
      poss = i * L + iota
      key = (ids << POS_BITS) + poss
      skey = jnp.sort(key)
      sid = skey >> POS_BITS
      spos = skey & POS_MASK
      key16[...] = sid
      nxt = plsc.load_gather(key16, [jnp.minimum(iota + 1, L - 1)])
      is_last = (sid != nxt) | (iota == L - 1)
      msk = is_last & (sid >= lo) & (sid < hi)
      idx = jnp.clip(sid - lo, 0, TS - 1)
      plsc.store_scatter(table_v, [idx], spos, mask=msk)
    return 0
  lax.fori_loop(0, NSCAN, scan, 0)

  pltpu.make_async_copy(ts_hbm, ts_v, sem_ts).wait()
  key16[...] = jnp.zeros((L,), jnp.int32)

  def emit(j, _):
    kvec = key16[...]
    lp = table_v[pl.ds(j * L, L)]
    upd = lp >= 0
    updi = jnp.where(upd, 1, 0)
    safe = jnp.maximum(lp, 0)
    tsg = plsc.load_gather(ts_v, [safe]) * updi.astype(jnp.float32)
    upd_loc[pl.ds(j * L, L)] = updi
    tsl_loc[pl.ds(j * L, L)] = tsg
    csum = plsc.cumsum(updi)
    offs = jnp.clip(kvec + csum - 1, 0, TS - 1)
    plsc.store_scatter(winp_v, [offs], safe, mask=upd)
    slot = lo + j * L + iota
    plsc.store_scatter(wins_v, [offs], slot, mask=upd)
    key16[...] = kvec + plsc.all_reduce_population_count(upd)
    return 0
  lax.fori_loop(0, nv, emit, 0)
  kvec = key16[...]
  k_s = kvec[0]

  @pl.when(k_s > 0)
  def _():
    kpad = ((k_s + WC - 1) // WC) * WC
    klast = jnp.full((L,), 0, jnp.int32) + (k_s - 1)
    lastw = plsc.load_gather(winp_v, [klast])
    lasts = plsc.load_gather(wins_v, [klast])

    def pad(t, _):
      idx = k_s + t * L + iota
      m = idx < kpad
      ii = jnp.clip(idx, 0, TS - 1)
      plsc.store_scatter(winp_v, [ii], lastw, mask=m)
      plsc.store_scatter(wins_v, [ii], lasts, mask=m)
      return 0
    lax.fori_loop(0, WC // L, pad, 0)

  pltpu.async_copy(winp_v, winp_hbm.at[wid], sem_ids)
  pltpu.async_copy(wins_v, wins_hbm.at[wid], sem_ts)

  def out_full():
    pltpu.sync_copy(upd_loc, upd_hbm.at[pl.ds(lo, TS)])
    pltpu.sync_copy(tsl_loc, aggts_hbm.at[pl.ds(lo, TS)])
  def out_tail():
    pltpu.sync_copy(upd_loc.at[pl.ds(0, TAIL)], upd_hbm.at[pl.ds(lo, TAIL)])
    pltpu.sync_copy(tsl_loc.at[pl.ds(0, TAIL)], aggts_hbm.at[pl.ds(lo, TAIL)])
  lax.cond(nv == NV_FULL, out_full, out_tail)

  key16[...] = kvec
  pltpu.sync_copy(key16, cnt_hbm.at[wid])
  pltpu.make_async_copy(winp_v, winp_hbm.at[wid], sem_ids).wait()
  pltpu.make_async_copy(wins_v, wins_hbm.at[wid], sem_ts).wait()


_sc_prep = pl.kernel(
    _prep_body,
    out_type=[
        jax.ShapeDtypeStruct((M,), jnp.int32),
        jax.ShapeDtypeStruct((M,), jnp.float32),
        jax.ShapeDtypeStruct((NW, TS), jnp.int32),
        jax.ShapeDtypeStruct((NW, TS), jnp.int32),
        jax.ShapeDtypeStruct((NW, L), jnp.int32),
    ],
    mesh=plsc.VectorSubcoreMesh(core_axis_name="c", subcore_axis_name="s"),
    compiler_params=pltpu.CompilerParams(needs_layout_passes=False),
    scratch_types=[
        pltpu.VMEM((B,), jnp.int32),
        pltpu.VMEM((B,), jnp.float32),
        pltpu.VMEM((TS,), jnp.int32),
        pltpu.VMEM((TS,), jnp.int32),
        pltpu.VMEM((TS,), jnp.float32),
        pltpu.VMEM((TS,), jnp.int32),
        pltpu.VMEM((TS,), jnp.int32),
        pltpu.VMEM((L,), jnp.int32),
        pltpu.SemaphoreType.DMA,
        pltpu.SemaphoreType.DMA,
    ],
)


def _scat_body(msgs_hbm, winp_hbm, wins_hbm, cnt_hbm, newmem_hbm,
               winp_v, wins_v, key16, src0_v, dst0_v, src1_v, dst1_v,
               src2_v, dst2_v, src3_v, dst3_v,
               rows0_v, rows1_v, rows2_v, rows3_v,
               g0, s0, g1, s1, g2, s2, g3, s3):
  wid = lax.axis_index("c") * NS + lax.axis_index("s")
  pltpu.async_copy(winp_hbm.at[wid], winp_v, g0)
  pltpu.async_copy(wins_hbm.at[wid], wins_v, g1)
  pltpu.sync_copy(cnt_hbm.at[wid], key16)
  k_s = key16[...][0]
  pltpu.make_async_copy(winp_hbm.at[wid], winp_v, g0).wait()
  pltpu.make_async_copy(wins_hbm.at[wid], wins_v, g1).wait()

  @pl.when(k_s > 0)
  def _():
    nch = (k_s + WC - 1) // WC

    def ldidx(c, sref, dref):
      def ld(t, _):
        gidx = c * WC + t * L + lax.iota(jnp.int32, L)
        sref[pl.ds(t * L, L)] = plsc.load_gather(winp_v, [gidx])
        dref[pl.ds(t * L, L)] = plsc.load_gather(wins_v, [gidx])
        return 0
      lax.fori_loop(0, WC // L, ld, 0)

    srefs = [src0_v, src1_v, src2_v, src3_v]
    drefs = [dst0_v, dst1_v, dst2_v, dst3_v]
    rrefs = [rows0_v, rows1_v, rows2_v, rows3_v]
    gsems = [g0, g1, g2, g3]
    ssems = [s0, s1, s2, s3]

    for i in range(4):
      @pl.when(i < nch)
      def _():
        ldidx(i, srefs[i], drefs[i])
        pltpu.async_copy(msgs_hbm.at[srefs[i]], rrefs[i], gsems[i])

    def wbody(u, _):
      cb = 4 * u
      for i in range(4):
        c = cb + i
        @pl.when(c < nch)
        def _():
          pltpu.make_async_copy(msgs_hbm.at[srefs[i]], rrefs[i],
                                gsems[i]).wait()
          pltpu.async_copy(rrefs[i], newmem_hbm.at[drefs[i]], ssems[i])
          @pl.when(c + 4 < nch)
          def _():
            pltpu.make_async_copy(rrefs[i], newmem_hbm.at[drefs[i]],
                                  ssems[i]).wait()
            ldidx(c + 4, srefs[i], drefs[i])
            pltpu.async_copy(msgs_hbm.at[srefs[i]], rrefs[i], gsems[i])
      return 0
    lax.fori_loop(0, (nch + 3) // 4, wbody, 0)
    for i in range(4):
      @pl.when(jnp.minimum(nch, 4) > i)
      def _():
        pltpu.make_async_copy(rrefs[i], newmem_hbm.at[drefs[i]],
                              ssems[i]).wait()


_sc_scatter = pl.kernel(
    _scat_body,
    out_type=[],
    mesh=plsc.VectorSubcoreMesh(core_axis_name="c", subcore_axis_name="s"),
    compiler_params=pltpu.CompilerParams(needs_layout_passes=False),
    scratch_types=[
        pltpu.VMEM((TS,), jnp.int32),
        pltpu.VMEM((TS,), jnp.int32),
        pltpu.VMEM((L,), jnp.int32),
        pltpu.VMEM((WC,), jnp.int32),
        pltpu.VMEM((WC,), jnp.int32),
        pltpu.VMEM((WC,), jnp.int32),
        pltpu.VMEM((WC,), jnp.int32),
        pltpu.VMEM((WC,), jnp.int32),
        pltpu.VMEM((WC,), jnp.int32),
        pltpu.VMEM((WC,), jnp.int32),
        pltpu.VMEM((WC,), jnp.int32),
        pltpu.VMEM((WC, D), jnp.float32),
        pltpu.VMEM((WC, D), jnp.float32),
        pltpu.VMEM((WC, D), jnp.float32),
        pltpu.VMEM((WC, D), jnp.float32),
        pltpu.SemaphoreType.DMA,
        pltpu.SemaphoreType.DMA,
        pltpu.SemaphoreType.DMA,
        pltpu.SemaphoreType.DMA,
        pltpu.SemaphoreType.DMA,
        pltpu.SemaphoreType.DMA,
        pltpu.SemaphoreType.DMA,
        pltpu.SemaphoreType.DMA,
    ],
)


def kernel(node_ids, messages, timestamps, mem):
  node_ids = node_ids.astype(jnp.int32)
  timestamps = timestamps.astype(jnp.float32)
  new_mem0 = _tc_copy(mem)
  upd, agg_ts, winp, wins, cnt = _sc_prep(node_ids, timestamps)
  r = jax.new_ref(new_mem0)
  _sc_scatter(messages, winp, wins, cnt, r)
  return r[...], upd.astype(bool), agg_ts
